# Initial kernel scaffold; baseline (speedup 1.0000x reference)
import functools

import jax
import jax.numpy as jnp
from jax import lax
from jax.experimental import pallas as pl
from jax.experimental.pallas import tpu as pltpu
from jax.experimental.pallas import tpu_sc as plsc

N = 10000
E = 160000
IN_FEATS = 256
N_HIDDEN = 512
N_CLASSES = 64
ALPHA = 0.1
K_STEPS = 10

NC = 2
NS = 16
NW = NC * NS
EPT = E // NW
CH = 125
NCH = EPT // CH
RPT = N // NS
DEG_W = 16


def _mlp_body(x_ref, w1_ref, b1_ref, w2_ref, b2_ref, w3_ref, b3_ref, o_ref):
    h = jnp.maximum(
        jnp.dot(x_ref[...], w1_ref[...], preferred_element_type=jnp.float32)
        + b1_ref[...], 0.0)
    h = jnp.maximum(
        jnp.dot(h, w2_ref[...], preferred_element_type=jnp.float32)
        + b2_ref[...], 0.0)
    o_ref[...] = (
        jnp.dot(h, w3_ref[...], preferred_element_type=jnp.float32)
        + b3_ref[...])


def _mlp(features, W1, b1, W2, b2, W3, b3):
    blk = 1000
    grid = N // blk
    full = lambda shape: pl.BlockSpec(shape, lambda i: (0, 0))
    return pl.pallas_call(
        _mlp_body,
        grid=(grid,),
        in_specs=[
            pl.BlockSpec((blk, IN_FEATS), lambda i: (i, 0)),
            full((IN_FEATS, N_HIDDEN)), full((1, N_HIDDEN)),
            full((N_HIDDEN, N_HIDDEN)), full((1, N_HIDDEN)),
            full((N_HIDDEN, N_CLASSES)), full((1, N_CLASSES)),
        ],
        out_specs=pl.BlockSpec((blk, N_CLASSES), lambda i: (i, 0)),
        out_shape=jax.ShapeDtypeStruct((N, N_CLASSES), jnp.float32),
    )(features, W1, b1.reshape(1, -1), W2, b2.reshape(1, -1),
      W3, b3.reshape(1, -1))


def _deg_body(dst_hbm, ones_hbm, zeros_hbm, out_hbm, dst_v, ones_v, deg_sh, sem):
    c = lax.axis_index("c")
    s = lax.axis_index("s")
    wid = c * NS + s
    pltpu.sync_copy(zeros_hbm, deg_sh.at[pl.ds(s * RPT, RPT)])
    pltpu.sync_copy(dst_hbm.at[wid], dst_v)
    pltpu.sync_copy(ones_hbm, ones_v)
    plsc.subcore_barrier()

    def step(j, _):
        pltpu.sync_copy(ones_v, deg_sh.at[dst_v.at[j]], add=True)
        return 0

    lax.fori_loop(0, NCH, step, 0)
    plsc.subcore_barrier()
    pltpu.sync_copy(deg_sh.at[pl.ds(s * RPT, RPT)],
                    out_hbm.at[c, pl.ds(s * RPT, RPT)])


def _degrees(dst3, ones, zeros):
    mesh = plsc.VectorSubcoreMesh(core_axis_name="c", subcore_axis_name="s")
    return pl.kernel(
        _deg_body,
        out_type=jax.ShapeDtypeStruct((NC, N, DEG_W), jnp.float32),
        mesh=mesh,
        scratch_types=[
            pltpu.VMEM((NCH, CH), jnp.int32),
            pltpu.VMEM((CH, DEG_W), jnp.float32),
            pltpu.VMEM_SHARED((N, DEG_W), jnp.float32),
            pltpu.SemaphoreType.DMA,
        ],
    )(dst3, ones, zeros)


def _scat_body(s_hbm, src_hbm, dst_hbm, zeros_hbm, out_hbm,
               src_v, dst_v, msg_v, agg_sh, sem):
    c = lax.axis_index("c")
    s = lax.axis_index("s")
    wid = c * NS + s
    pltpu.sync_copy(zeros_hbm, agg_sh.at[pl.ds(s * RPT, RPT)])
    pltpu.sync_copy(src_hbm.at[wid], src_v)
    pltpu.sync_copy(dst_hbm.at[wid], dst_v)
    plsc.subcore_barrier()

    def step(j, _):
        pltpu.async_copy(s_hbm.at[src_v.at[j]], msg_v, sem).wait()
        pltpu.sync_copy(msg_v, agg_sh.at[dst_v.at[j]], add=True)
        return 0

    lax.fori_loop(0, NCH, step, 0)
    plsc.subcore_barrier()
    pltpu.sync_copy(agg_sh.at[pl.ds(s * RPT, RPT)],
                    out_hbm.at[c, pl.ds(s * RPT, RPT)])


def _scatter(s_cur, src3, dst3, zeros):
    mesh = plsc.VectorSubcoreMesh(core_axis_name="c", subcore_axis_name="s")
    return pl.kernel(
        _scat_body,
        out_type=jax.ShapeDtypeStruct((NC, N, N_CLASSES), jnp.float32),
        mesh=mesh,
        scratch_types=[
            pltpu.VMEM((NCH, CH), jnp.int32),
            pltpu.VMEM((NCH, CH), jnp.int32),
            pltpu.VMEM((CH, N_CLASSES), jnp.float32),
            pltpu.VMEM_SHARED((N, N_CLASSES), jnp.float32),
            pltpu.SemaphoreType.DMA,
        ],
    )(s_cur, src3, dst3, zeros)


def _prep_body(dp_ref, h_ref, nrm_ref, s0_ref):
    deg = dp_ref[0, :, 0:1] + dp_ref[1, :, 0:1]
    nrm = lax.rsqrt(jnp.maximum(deg, 1.0))
    nrm_ref[...] = jnp.broadcast_to(nrm, nrm_ref.shape)
    s0_ref[...] = nrm_ref[...] * h_ref[...]


def _prep(dp, h):
    blk = 1000
    return pl.pallas_call(
        _prep_body,
        grid=(N // blk,),
        in_specs=[
            pl.BlockSpec((NC, blk, DEG_W), lambda i: (0, i, 0)),
            pl.BlockSpec((blk, N_CLASSES), lambda i: (i, 0)),
        ],
        out_specs=[
            pl.BlockSpec((blk, N_CLASSES), lambda i: (i, 0)),
            pl.BlockSpec((blk, N_CLASSES), lambda i: (i, 0)),
        ],
        out_shape=[
            jax.ShapeDtypeStruct((N, N_CLASSES), jnp.float32),
            jax.ShapeDtypeStruct((N, N_CLASSES), jnp.float32),
        ],
    )(dp, h)


def _upd_body(agg_ref, nrm_ref, h_ref, o_ref, *, last):
    agg = agg_ref[0] + agg_ref[1]
    nrm = nrm_ref[...]
    if last:
        o_ref[...] = (1.0 - ALPHA) * nrm * agg + ALPHA * h_ref[...]
    else:
        o_ref[...] = (1.0 - ALPHA) * nrm * nrm * agg + ALPHA * nrm * h_ref[...]


def _update(agg, nrm, h, last):
    blk = 1000
    return pl.pallas_call(
        functools.partial(_upd_body, last=last),
        grid=(N // blk,),
        in_specs=[
            pl.BlockSpec((NC, blk, N_CLASSES), lambda i: (0, i, 0)),
            pl.BlockSpec((blk, N_CLASSES), lambda i: (i, 0)),
            pl.BlockSpec((blk, N_CLASSES), lambda i: (i, 0)),
        ],
        out_specs=pl.BlockSpec((blk, N_CLASSES), lambda i: (i, 0)),
        out_shape=jax.ShapeDtypeStruct((N, N_CLASSES), jnp.float32),
    )(agg, nrm, h)


def kernel(features, edge_index, W1, b1, W2, b2, W3, b3):
    src3 = edge_index[0].reshape(NW, NCH, CH)
    dst3 = edge_index[1].reshape(NW, NCH, CH)
    ones = jnp.ones((CH, DEG_W), jnp.float32)
    zeros_deg = jnp.zeros((RPT, DEG_W), jnp.float32)
    zeros_agg = jnp.zeros((RPT, N_CLASSES), jnp.float32)

    h = _mlp(features, W1, b1, W2, b2, W3, b3)
    dp = _degrees(dst3, ones, zeros_deg)
    nrm, s_cur = _prep(dp, h)
    for t in range(K_STEPS):
        agg = _scatter(s_cur, src3, dst3, zeros_agg)
        s_cur = _update(agg, nrm, h, last=(t == K_STEPS - 1))
    return s_cur


# speedup vs baseline: 6.7766x; 6.7766x over previous
import functools

import jax
import jax.numpy as jnp
from jax import lax
from jax.experimental import pallas as pl
from jax.experimental.pallas import tpu as pltpu
from jax.experimental.pallas import tpu_sc as plsc

N = 10000
E = 160000
IN_FEATS = 256
N_HIDDEN = 512
N_CLASSES = 64
ALPHA = 0.1
K_STEPS = 10

NC = 2
NS = 16
NW = NC * NS
EPT = E // NW
CH = 125
NCH = EPT // CH
NPAD = 10240
RPT = NPAD // NS
DEG_W = 16


def _mlp_body(x_ref, w1_ref, b1_ref, w2_ref, b2_ref, w3_ref, b3_ref, o_ref):
    h = jnp.maximum(
        jnp.dot(x_ref[...], w1_ref[...], preferred_element_type=jnp.float32)
        + b1_ref[...], 0.0)
    h = jnp.maximum(
        jnp.dot(h, w2_ref[...], preferred_element_type=jnp.float32)
        + b2_ref[...], 0.0)
    o_ref[...] = (
        jnp.dot(h, w3_ref[...], preferred_element_type=jnp.float32)
        + b3_ref[...])


def _mlp(features, W1, b1, W2, b2, W3, b3):
    blk = 1000
    grid = N // blk
    full = lambda shape: pl.BlockSpec(shape, lambda i: (0, 0))
    return pl.pallas_call(
        _mlp_body,
        grid=(grid,),
        in_specs=[
            pl.BlockSpec((blk, IN_FEATS), lambda i: (i, 0)),
            full((IN_FEATS, N_HIDDEN)), full((1, N_HIDDEN)),
            full((N_HIDDEN, N_HIDDEN)), full((1, N_HIDDEN)),
            full((N_HIDDEN, N_CLASSES)), full((1, N_CLASSES)),
        ],
        out_specs=pl.BlockSpec((blk, N_CLASSES), lambda i: (i, 0)),
        out_shape=jax.ShapeDtypeStruct((N, N_CLASSES), jnp.float32),
    )(features, W1, b1.reshape(1, -1), W2, b2.reshape(1, -1),
      W3, b3.reshape(1, -1))


def _deg_body(dst_hbm, ones_hbm, zeros_hbm, out_hbm, dst_v, ones_v, deg_sh, sem):
    c = lax.axis_index("c")
    s = lax.axis_index("s")
    wid = c * NS + s
    pltpu.sync_copy(zeros_hbm, deg_sh.at[pl.ds(s * RPT, RPT)])
    pltpu.sync_copy(dst_hbm.at[wid], dst_v)
    pltpu.sync_copy(ones_hbm, ones_v)
    plsc.subcore_barrier()

    def step(j, _):
        pltpu.sync_copy(ones_v, deg_sh.at[dst_v.at[j]], add=True)
        return 0

    lax.fori_loop(0, NCH, step, 0)
    plsc.subcore_barrier()
    pltpu.sync_copy(deg_sh.at[pl.ds(s * RPT, RPT)],
                    out_hbm.at[c, pl.ds(s * RPT, RPT)])


def _degrees(dst3, ones, zeros):
    mesh = plsc.VectorSubcoreMesh(core_axis_name="c", subcore_axis_name="s",
                                  num_cores=NC, num_subcores=NS)
    return pl.kernel(
        _deg_body,
        out_type=jax.ShapeDtypeStruct((NC, NPAD, DEG_W), jnp.float32),
        mesh=mesh,
        compiler_params=pltpu.CompilerParams(use_tc_tiling_on_sc=False),
        scratch_types=[
            pltpu.VMEM((NCH, CH), jnp.int32),
            pltpu.VMEM((CH, DEG_W), jnp.float32),
            pltpu.VMEM_SHARED((NPAD, DEG_W), jnp.float32),
            pltpu.SemaphoreType.DMA,
        ],
    )(dst3, ones, zeros)


def _scat_body(s_hbm, src_hbm, dst_hbm, zeros_hbm, out_hbm,
               src_v, dst_v, msg_v, agg_sh, sem):
    c = lax.axis_index("c")
    s = lax.axis_index("s")
    wid = c * NS + s
    pltpu.sync_copy(zeros_hbm, agg_sh.at[pl.ds(s * RPT, RPT)])
    pltpu.sync_copy(src_hbm.at[wid], src_v)
    pltpu.sync_copy(dst_hbm.at[wid], dst_v)
    plsc.subcore_barrier()

    def step(j, _):
        pltpu.async_copy(s_hbm.at[src_v.at[j]], msg_v, sem).wait()
        pltpu.sync_copy(msg_v, agg_sh.at[dst_v.at[j]], add=True)
        return 0

    lax.fori_loop(0, NCH, step, 0)
    plsc.subcore_barrier()
    pltpu.sync_copy(agg_sh.at[pl.ds(s * RPT, RPT)],
                    out_hbm.at[c, pl.ds(s * RPT, RPT)])


def _scatter(s_cur, src3, dst3, zeros):
    mesh = plsc.VectorSubcoreMesh(core_axis_name="c", subcore_axis_name="s",
                                  num_cores=NC, num_subcores=NS)
    return pl.kernel(
        _scat_body,
        out_type=jax.ShapeDtypeStruct((NC, NPAD, N_CLASSES), jnp.float32),
        mesh=mesh,
        compiler_params=pltpu.CompilerParams(use_tc_tiling_on_sc=False),
        scratch_types=[
            pltpu.VMEM((NCH, CH), jnp.int32),
            pltpu.VMEM((NCH, CH), jnp.int32),
            pltpu.VMEM((CH, N_CLASSES), jnp.float32),
            pltpu.VMEM_SHARED((NPAD, N_CLASSES), jnp.float32),
            pltpu.SemaphoreType.DMA,
        ],
    )(s_cur, src3, dst3, zeros)


def _prep_body(dp_ref, h_ref, nrm_ref, s0_ref):
    deg = dp_ref[0, :, 0:1] + dp_ref[1, :, 0:1]
    nrm = lax.rsqrt(jnp.maximum(deg, 1.0))
    nrm_ref[...] = jnp.broadcast_to(nrm, nrm_ref.shape)
    s0_ref[...] = nrm_ref[...] * h_ref[...]


def _prep(dp, h):
    blk = 1000
    return pl.pallas_call(
        _prep_body,
        grid=(N // blk,),
        in_specs=[
            pl.BlockSpec((NC, blk, DEG_W), lambda i: (0, i, 0)),
            pl.BlockSpec((blk, N_CLASSES), lambda i: (i, 0)),
        ],
        out_specs=[
            pl.BlockSpec((blk, N_CLASSES), lambda i: (i, 0)),
            pl.BlockSpec((blk, N_CLASSES), lambda i: (i, 0)),
        ],
        out_shape=[
            jax.ShapeDtypeStruct((N, N_CLASSES), jnp.float32),
            jax.ShapeDtypeStruct((N, N_CLASSES), jnp.float32),
        ],
    )(dp, h)


def _upd_body(agg_ref, nrm_ref, h_ref, o_ref, *, last):
    agg = agg_ref[0] + agg_ref[1]
    nrm = nrm_ref[...]
    if last:
        o_ref[...] = (1.0 - ALPHA) * nrm * agg + ALPHA * h_ref[...]
    else:
        o_ref[...] = (1.0 - ALPHA) * nrm * nrm * agg + ALPHA * nrm * h_ref[...]


def _update(agg, nrm, h, last):
    blk = 1000
    return pl.pallas_call(
        functools.partial(_upd_body, last=last),
        grid=(N // blk,),
        in_specs=[
            pl.BlockSpec((NC, blk, N_CLASSES), lambda i: (0, i, 0)),
            pl.BlockSpec((blk, N_CLASSES), lambda i: (i, 0)),
            pl.BlockSpec((blk, N_CLASSES), lambda i: (i, 0)),
        ],
        out_specs=pl.BlockSpec((blk, N_CLASSES), lambda i: (i, 0)),
        out_shape=jax.ShapeDtypeStruct((N, N_CLASSES), jnp.float32),
    )(agg, nrm, h)


def kernel(features, edge_index, W1, b1, W2, b2, W3, b3):
    src3 = edge_index[0].reshape(NW, NCH, CH)
    dst3 = edge_index[1].reshape(NW, NCH, CH)
    ones = jnp.ones((CH, DEG_W), jnp.float32)
    zeros_deg = jnp.zeros((RPT, DEG_W), jnp.float32)
    zeros_agg = jnp.zeros((RPT, N_CLASSES), jnp.float32)

    h = _mlp(features, W1, b1, W2, b2, W3, b3)
    dp = _degrees(dst3, ones, zeros_deg)
    nrm, s_cur = _prep(dp, h)
    for t in range(K_STEPS):
        agg = _scatter(s_cur, src3, dst3, zeros_agg)
        s_cur = _update(agg, nrm, h, last=(t == K_STEPS - 1))
    return s_cur


# speedup vs baseline: 8.9386x; 1.3190x over previous
import functools

import jax
import jax.numpy as jnp
from jax import lax
from jax.experimental import pallas as pl
from jax.experimental.pallas import tpu as pltpu
from jax.experimental.pallas import tpu_sc as plsc

N = 10000
E = 160000
IN_FEATS = 256
N_HIDDEN = 512
N_CLASSES = 64
ALPHA = 0.1
K_STEPS = 10

NC = 2
NS = 16
NW = NC * NS
EPT = E // NW
CH = 125
NCH = EPT // CH
NPAD = 10240
RPT = NPAD // NS
DEG_W = 16


def _mlp_body(x_ref, w1_ref, b1_ref, w2_ref, b2_ref, w3_ref, b3_ref, o_ref):
    h = jnp.maximum(
        jnp.dot(x_ref[...], w1_ref[...], preferred_element_type=jnp.float32)
        + b1_ref[...], 0.0)
    h = jnp.maximum(
        jnp.dot(h, w2_ref[...], preferred_element_type=jnp.float32)
        + b2_ref[...], 0.0)
    o_ref[...] = (
        jnp.dot(h, w3_ref[...], preferred_element_type=jnp.float32)
        + b3_ref[...])


def _mlp(features, W1, b1, W2, b2, W3, b3):
    blk = 1000
    grid = N // blk
    full = lambda shape: pl.BlockSpec(shape, lambda i: (0, 0))
    return pl.pallas_call(
        _mlp_body,
        grid=(grid,),
        in_specs=[
            pl.BlockSpec((blk, IN_FEATS), lambda i: (i, 0)),
            full((IN_FEATS, N_HIDDEN)), full((1, N_HIDDEN)),
            full((N_HIDDEN, N_HIDDEN)), full((1, N_HIDDEN)),
            full((N_HIDDEN, N_CLASSES)), full((1, N_CLASSES)),
        ],
        out_specs=pl.BlockSpec((blk, N_CLASSES), lambda i: (i, 0)),
        out_shape=jax.ShapeDtypeStruct((N, N_CLASSES), jnp.float32),
    )(features, W1, b1.reshape(1, -1), W2, b2.reshape(1, -1),
      W3, b3.reshape(1, -1))


def _deg_body(dst_hbm, ones_hbm, zeros_hbm, out_hbm, dst_v, ones_v, deg_sh, sem):
    c = lax.axis_index("c")
    s = lax.axis_index("s")
    wid = c * NS + s
    pltpu.sync_copy(zeros_hbm, deg_sh.at[pl.ds(s * RPT, RPT)])
    pltpu.sync_copy(dst_hbm.at[wid], dst_v)
    pltpu.sync_copy(ones_hbm, ones_v)
    plsc.subcore_barrier()

    def step(j, _):
        pltpu.sync_copy(ones_v, deg_sh.at[dst_v.at[j]], add=True)
        return 0

    lax.fori_loop(0, NCH, step, 0)
    plsc.subcore_barrier()
    pltpu.sync_copy(deg_sh.at[pl.ds(s * RPT, RPT)],
                    out_hbm.at[c, pl.ds(s * RPT, RPT)])


def _degrees(dst3, ones, zeros):
    mesh = plsc.VectorSubcoreMesh(core_axis_name="c", subcore_axis_name="s",
                                  num_cores=NC, num_subcores=NS)
    return pl.kernel(
        _deg_body,
        out_type=jax.ShapeDtypeStruct((NC, NPAD, DEG_W), jnp.float32),
        mesh=mesh,
        compiler_params=pltpu.CompilerParams(use_tc_tiling_on_sc=False),
        scratch_types=[
            pltpu.VMEM((NCH, CH), jnp.int32),
            pltpu.VMEM((CH, DEG_W), jnp.float32),
            pltpu.VMEM_SHARED((NPAD, DEG_W), jnp.float32),
            pltpu.SemaphoreType.DMA,
        ],
    )(dst3, ones, zeros)


def _scat_body(s_hbm, src_hbm, dst_hbm, zeros_hbm, out_hbm,
               src_v, dst_v, msg_v, agg_sh, sem0, sem1):
    c = lax.axis_index("c")
    s = lax.axis_index("s")
    wid = c * NS + s
    pltpu.sync_copy(zeros_hbm, agg_sh.at[pl.ds(s * RPT, RPT)])
    pltpu.sync_copy(src_hbm.at[wid], src_v)
    pltpu.sync_copy(dst_hbm.at[wid], dst_v)
    plsc.subcore_barrier()

    npair = NCH // 2
    pltpu.async_copy(s_hbm.at[src_v.at[0]], msg_v.at[0], sem0)

    def step(i, _):
        a = 2 * i
        b = a + 1
        pltpu.async_copy(s_hbm.at[src_v.at[b]], msg_v.at[1], sem1)
        pltpu.make_async_copy(s_hbm.at[src_v.at[a]], msg_v.at[0], sem0).wait()
        pltpu.sync_copy(msg_v.at[0], agg_sh.at[dst_v.at[a]], add=True)

        @pl.when(i + 1 < npair)
        def _():
            pltpu.async_copy(s_hbm.at[src_v.at[a + 2]], msg_v.at[0], sem0)

        pltpu.make_async_copy(s_hbm.at[src_v.at[b]], msg_v.at[1], sem1).wait()
        pltpu.sync_copy(msg_v.at[1], agg_sh.at[dst_v.at[b]], add=True)
        return 0

    lax.fori_loop(0, npair, step, 0)
    plsc.subcore_barrier()
    pltpu.sync_copy(agg_sh.at[pl.ds(s * RPT, RPT)],
                    out_hbm.at[c, pl.ds(s * RPT, RPT)])


def _scatter(s_cur, src3, dst3, zeros):
    mesh = plsc.VectorSubcoreMesh(core_axis_name="c", subcore_axis_name="s",
                                  num_cores=NC, num_subcores=NS)
    return pl.kernel(
        _scat_body,
        out_type=jax.ShapeDtypeStruct((NC, NPAD, N_CLASSES), jnp.float32),
        mesh=mesh,
        compiler_params=pltpu.CompilerParams(use_tc_tiling_on_sc=False),
        scratch_types=[
            pltpu.VMEM((NCH, CH), jnp.int32),
            pltpu.VMEM((NCH, CH), jnp.int32),
            pltpu.VMEM((2, CH, N_CLASSES), jnp.float32),
            pltpu.VMEM_SHARED((NPAD, N_CLASSES), jnp.float32),
            pltpu.SemaphoreType.DMA,
            pltpu.SemaphoreType.DMA,
        ],
    )(s_cur, src3, dst3, zeros)


def _prep_body(dp_ref, h_ref, nrm_ref, s0_ref):
    deg = dp_ref[0, :, 0:1] + dp_ref[1, :, 0:1]
    nrm = lax.rsqrt(jnp.maximum(deg, 1.0))
    nrm_ref[...] = jnp.broadcast_to(nrm, nrm_ref.shape)
    s0_ref[...] = nrm_ref[...] * h_ref[...]


def _prep(dp, h):
    blk = 1000
    return pl.pallas_call(
        _prep_body,
        grid=(N // blk,),
        in_specs=[
            pl.BlockSpec((NC, blk, DEG_W), lambda i: (0, i, 0)),
            pl.BlockSpec((blk, N_CLASSES), lambda i: (i, 0)),
        ],
        out_specs=[
            pl.BlockSpec((blk, N_CLASSES), lambda i: (i, 0)),
            pl.BlockSpec((blk, N_CLASSES), lambda i: (i, 0)),
        ],
        out_shape=[
            jax.ShapeDtypeStruct((N, N_CLASSES), jnp.float32),
            jax.ShapeDtypeStruct((N, N_CLASSES), jnp.float32),
        ],
    )(dp, h)


def _upd_body(agg_ref, nrm_ref, h_ref, o_ref, *, last):
    agg = agg_ref[0] + agg_ref[1]
    nrm = nrm_ref[...]
    if last:
        o_ref[...] = (1.0 - ALPHA) * nrm * agg + ALPHA * h_ref[...]
    else:
        o_ref[...] = (1.0 - ALPHA) * nrm * nrm * agg + ALPHA * nrm * h_ref[...]


def _update(agg, nrm, h, last):
    blk = 1000
    return pl.pallas_call(
        functools.partial(_upd_body, last=last),
        grid=(N // blk,),
        in_specs=[
            pl.BlockSpec((NC, blk, N_CLASSES), lambda i: (0, i, 0)),
            pl.BlockSpec((blk, N_CLASSES), lambda i: (i, 0)),
            pl.BlockSpec((blk, N_CLASSES), lambda i: (i, 0)),
        ],
        out_specs=pl.BlockSpec((blk, N_CLASSES), lambda i: (i, 0)),
        out_shape=jax.ShapeDtypeStruct((N, N_CLASSES), jnp.float32),
    )(agg, nrm, h)


def kernel(features, edge_index, W1, b1, W2, b2, W3, b3):
    src3 = edge_index[0].reshape(NW, NCH, CH)
    dst3 = edge_index[1].reshape(NW, NCH, CH)
    ones = jnp.ones((CH, DEG_W), jnp.float32)
    zeros_deg = jnp.zeros((RPT, DEG_W), jnp.float32)
    zeros_agg = jnp.zeros((RPT, N_CLASSES), jnp.float32)

    h = _mlp(features, W1, b1, W2, b2, W3, b3)
    dp = _degrees(dst3, ones, zeros_deg)
    nrm, s_cur = _prep(dp, h)
    for t in range(K_STEPS):
        agg = _scatter(s_cur, src3, dst3, zeros_agg)
        s_cur = _update(agg, nrm, h, last=(t == K_STEPS - 1))
    return s_cur


# speedup vs baseline: 9.9570x; 1.1139x over previous
import functools

import jax
import jax.numpy as jnp
from jax import lax
from jax.experimental import pallas as pl
from jax.experimental.pallas import tpu as pltpu
from jax.experimental.pallas import tpu_sc as plsc

N = 10000
E = 160000
IN_FEATS = 256
N_HIDDEN = 512
N_CLASSES = 64
ALPHA = 0.1
K_STEPS = 10

NC = 2
NS = 16
NW = NC * NS
EPT = E // NW
CH = 125
NCH = EPT // CH
NPAD = 10240
RPT = NPAD // NS
DEG_W = 16


def _mlp_body(x_ref, w1_ref, b1_ref, w2_ref, b2_ref, w3_ref, b3_ref, o_ref):
    bf = jnp.bfloat16
    h = jnp.maximum(
        jnp.dot(x_ref[...].astype(bf), w1_ref[...].astype(bf),
                preferred_element_type=jnp.float32) + b1_ref[...], 0.0)
    h = jnp.maximum(
        jnp.dot(h.astype(bf), w2_ref[...].astype(bf),
                preferred_element_type=jnp.float32) + b2_ref[...], 0.0)
    o_ref[...] = (
        jnp.dot(h.astype(bf), w3_ref[...].astype(bf),
                preferred_element_type=jnp.float32) + b3_ref[...])


def _mlp(features, W1, b1, W2, b2, W3, b3):
    blk = 1000
    grid = N // blk
    full = lambda shape: pl.BlockSpec(shape, lambda i: (0, 0))
    return pl.pallas_call(
        _mlp_body,
        grid=(grid,),
        in_specs=[
            pl.BlockSpec((blk, IN_FEATS), lambda i: (i, 0)),
            full((IN_FEATS, N_HIDDEN)), full((1, N_HIDDEN)),
            full((N_HIDDEN, N_HIDDEN)), full((1, N_HIDDEN)),
            full((N_HIDDEN, N_CLASSES)), full((1, N_CLASSES)),
        ],
        out_specs=pl.BlockSpec((blk, N_CLASSES), lambda i: (i, 0)),
        out_shape=jax.ShapeDtypeStruct((N, N_CLASSES), jnp.float32),
    )(features, W1, b1.reshape(1, -1), W2, b2.reshape(1, -1),
      W3, b3.reshape(1, -1))


def _deg_body(dst_hbm, ones_hbm, zeros_hbm, out_hbm, dst_v, ones_v, deg_sh, sem):
    c = lax.axis_index("c")
    s = lax.axis_index("s")
    wid = c * NS + s
    pltpu.sync_copy(zeros_hbm, deg_sh.at[pl.ds(s * RPT, RPT)])
    pltpu.sync_copy(dst_hbm.at[wid], dst_v)
    pltpu.sync_copy(ones_hbm, ones_v)
    plsc.subcore_barrier()

    def step(j, _):
        pltpu.sync_copy(ones_v, deg_sh.at[dst_v.at[j]], add=True)
        return 0

    lax.fori_loop(0, NCH, step, 0)
    plsc.subcore_barrier()
    pltpu.sync_copy(deg_sh.at[pl.ds(s * RPT, RPT)],
                    out_hbm.at[c, pl.ds(s * RPT, RPT)])


def _degrees(dst3, ones, zeros):
    mesh = plsc.VectorSubcoreMesh(core_axis_name="c", subcore_axis_name="s",
                                  num_cores=NC, num_subcores=NS)
    return pl.kernel(
        _deg_body,
        out_type=jax.ShapeDtypeStruct((NC, NPAD, DEG_W), jnp.float32),
        mesh=mesh,
        compiler_params=pltpu.CompilerParams(use_tc_tiling_on_sc=False),
        scratch_types=[
            pltpu.VMEM((NCH, CH), jnp.int32),
            pltpu.VMEM((CH, DEG_W), jnp.float32),
            pltpu.VMEM_SHARED((NPAD, DEG_W), jnp.float32),
            pltpu.SemaphoreType.DMA,
        ],
    )(dst3, ones, zeros)


NBUF = 4


def _scat_body(s_hbm, src_hbm, dst_hbm, zeros_hbm, out_hbm,
               src_v, dst_v, msg_v, agg_sh, *sems):
    c = lax.axis_index("c")
    s = lax.axis_index("s")
    wid = c * NS + s
    pltpu.sync_copy(zeros_hbm, agg_sh.at[pl.ds(s * RPT, RPT)])
    pltpu.sync_copy(src_hbm.at[wid], src_v)
    pltpu.sync_copy(dst_hbm.at[wid], dst_v)
    plsc.subcore_barrier()

    for b in range(NBUF - 1):
        pltpu.async_copy(s_hbm.at[src_v.at[b]], msg_v.at[b], sems[b])

    def step(g, _):
        for b in range(NBUF):
            j = g * NBUF + b
            nb = (b + NBUF - 1) % NBUF

            @pl.when(j + NBUF - 1 < NCH)
            def _():
                pltpu.async_copy(s_hbm.at[src_v.at[j + NBUF - 1]],
                                 msg_v.at[nb], sems[nb])

            pltpu.make_async_copy(s_hbm.at[src_v.at[j]], msg_v.at[b],
                                  sems[b]).wait()
            pltpu.sync_copy(msg_v.at[b], agg_sh.at[dst_v.at[j]], add=True)
        return 0

    lax.fori_loop(0, NCH // NBUF, step, 0)
    plsc.subcore_barrier()
    pltpu.sync_copy(agg_sh.at[pl.ds(s * RPT, RPT)],
                    out_hbm.at[c, pl.ds(s * RPT, RPT)])


def _scatter(s_cur, src3, dst3, zeros):
    mesh = plsc.VectorSubcoreMesh(core_axis_name="c", subcore_axis_name="s",
                                  num_cores=NC, num_subcores=NS)
    return pl.kernel(
        _scat_body,
        out_type=jax.ShapeDtypeStruct((NC, NPAD, N_CLASSES), jnp.float32),
        mesh=mesh,
        compiler_params=pltpu.CompilerParams(use_tc_tiling_on_sc=False),
        scratch_types=[
            pltpu.VMEM((NCH, CH), jnp.int32),
            pltpu.VMEM((NCH, CH), jnp.int32),
            pltpu.VMEM((NBUF, CH, N_CLASSES), jnp.float32),
            pltpu.VMEM_SHARED((NPAD, N_CLASSES), jnp.float32),
        ] + [pltpu.SemaphoreType.DMA] * NBUF,
    )(s_cur, src3, dst3, zeros)


def _prep_body(dp_ref, h_ref, nrm_ref, s0_ref):
    deg = dp_ref[0, :, 0:1] + dp_ref[1, :, 0:1]
    nrm = lax.rsqrt(jnp.maximum(deg, 1.0))
    nrm_ref[...] = jnp.broadcast_to(nrm, nrm_ref.shape)
    s0_ref[...] = nrm_ref[...] * h_ref[...]


def _prep(dp, h):
    blk = 1000
    return pl.pallas_call(
        _prep_body,
        grid=(N // blk,),
        in_specs=[
            pl.BlockSpec((NC, blk, DEG_W), lambda i: (0, i, 0)),
            pl.BlockSpec((blk, N_CLASSES), lambda i: (i, 0)),
        ],
        out_specs=[
            pl.BlockSpec((blk, N_CLASSES), lambda i: (i, 0)),
            pl.BlockSpec((blk, N_CLASSES), lambda i: (i, 0)),
        ],
        out_shape=[
            jax.ShapeDtypeStruct((N, N_CLASSES), jnp.float32),
            jax.ShapeDtypeStruct((N, N_CLASSES), jnp.float32),
        ],
    )(dp, h)


def _upd_body(agg_ref, nrm_ref, h_ref, o_ref, *, last):
    agg = agg_ref[0] + agg_ref[1]
    nrm = nrm_ref[...]
    if last:
        o_ref[...] = (1.0 - ALPHA) * nrm * agg + ALPHA * h_ref[...]
    else:
        o_ref[...] = (1.0 - ALPHA) * nrm * nrm * agg + ALPHA * nrm * h_ref[...]


def _update(agg, nrm, h, last):
    blk = 1000
    return pl.pallas_call(
        functools.partial(_upd_body, last=last),
        grid=(N // blk,),
        in_specs=[
            pl.BlockSpec((NC, blk, N_CLASSES), lambda i: (0, i, 0)),
            pl.BlockSpec((blk, N_CLASSES), lambda i: (i, 0)),
            pl.BlockSpec((blk, N_CLASSES), lambda i: (i, 0)),
        ],
        out_specs=pl.BlockSpec((blk, N_CLASSES), lambda i: (i, 0)),
        out_shape=jax.ShapeDtypeStruct((N, N_CLASSES), jnp.float32),
    )(agg, nrm, h)


def kernel(features, edge_index, W1, b1, W2, b2, W3, b3):
    src3 = edge_index[0].reshape(NW, NCH, CH)
    dst3 = edge_index[1].reshape(NW, NCH, CH)
    ones = jnp.ones((CH, DEG_W), jnp.float32)
    zeros_deg = jnp.zeros((RPT, DEG_W), jnp.float32)
    zeros_agg = jnp.zeros((RPT, N_CLASSES), jnp.float32)

    h = _mlp(features, W1, b1, W2, b2, W3, b3)
    dp = _degrees(dst3, ones, zeros_deg)
    nrm, s_cur = _prep(dp, h)
    for t in range(K_STEPS):
        agg = _scatter(s_cur, src3, dst3, zeros_agg)
        s_cur = _update(agg, nrm, h, last=(t == K_STEPS - 1))
    return s_cur
